# initial kernel scaffold (unmeasured)
import jax
import jax.numpy as jnp
from jax import lax
from jax.experimental import pallas as pl
from jax.experimental.pallas import tpu as pltpu


def kernel(
    x,
):
    def body(*refs):
        pass

    out_shape = jax.ShapeDtypeStruct(..., jnp.float32)
    return pl.pallas_call(body, out_shape=out_shape)(...)


# baseline (device time: 31044 ns/iter reference)
import jax
import jax.numpy as jnp
from jax import lax
from jax.experimental import pallas as pl
from jax.experimental.pallas import tpu as pltpu


def kernel(x):
    m, n = x.shape

    def body(x_ref, out_ref, send_buf, recv_buf, send_sem, recv_sem):
        my_x = lax.axis_index("x")
        my_y = lax.axis_index("y")
        my_z = lax.axis_index("z")
        peer = (my_x, 1 - my_y, my_z)

        barrier_sem = pltpu.get_barrier_semaphore()
        pl.semaphore_signal(
            barrier_sem, inc=1, device_id=peer,
            device_id_type=pl.DeviceIdType.MESH,
        )
        pl.semaphore_wait(barrier_sem, 1)

        send_buf[...] = x_ref[...].astype(jnp.bfloat16)
        rdma = pltpu.make_async_remote_copy(
            src_ref=send_buf,
            dst_ref=recv_buf,
            send_sem=send_sem,
            recv_sem=recv_sem,
            device_id=peer,
            device_id_type=pl.DeviceIdType.MESH,
        )
        rdma.start()
        rdma.wait()
        out_ref[...] = x_ref[...] + recv_buf[...].astype(jnp.float32)

    return pl.pallas_call(
        body,
        out_shape=jax.ShapeDtypeStruct((m, n), x.dtype),
        in_specs=[pl.BlockSpec(memory_space=pltpu.VMEM)],
        out_specs=pl.BlockSpec(memory_space=pltpu.VMEM),
        scratch_shapes=[
            pltpu.VMEM((m, n), jnp.bfloat16),
            pltpu.VMEM((m, n), jnp.bfloat16),
            pltpu.SemaphoreType.DMA,
            pltpu.SemaphoreType.DMA,
        ],
        compiler_params=pltpu.CompilerParams(collective_id=0),
    )(x)


# device time: 28746 ns/iter; 1.0799x vs baseline; 1.0799x over previous
import jax
import jax.numpy as jnp
from jax import lax
from jax.experimental import pallas as pl
from jax.experimental.pallas import tpu as pltpu


def kernel(x):
    m, n = x.shape
    mb = m // 4

    def body(x_ref, out_ref, raw, yrecv, summ, xrecv, zrecv, drecv,
             ssems, rsems):
        mx = lax.axis_index("x")
        my = lax.axis_index("y")
        mz = lax.axis_index("z")
        y_peer = (mx, 1 - my, mz)
        x_peer = (1 - mx, my, mz)
        z_peer = (mx, my, 1 - mz)
        d_peer = (1 - mx, my, 1 - mz)
        b_mine = 2 * mx + mz
        b_x = 2 * (1 - mx) + mz
        b_z = 2 * mx + (1 - mz)
        b_d = 2 * (1 - mx) + (1 - mz)

        barrier_sem = pltpu.get_barrier_semaphore()
        for p in (y_peer, x_peer, z_peer, d_peer):
            pl.semaphore_signal(
                barrier_sem, inc=1, device_id=p,
                device_id_type=pl.DeviceIdType.MESH,
            )
        pl.semaphore_wait(barrier_sem, 4)

        def exchange(src, dst, i, peer):
            return pltpu.make_async_remote_copy(
                src_ref=src, dst_ref=dst,
                send_sem=ssems.at[i], recv_sem=rsems.at[i],
                device_id=peer, device_id_type=pl.DeviceIdType.MESH,
            )

        raw[...] = x_ref[pl.ds(b_mine * mb, mb), :].astype(jnp.bfloat16)
        y_rdma = exchange(raw, yrecv, 0, y_peer)
        y_rdma.start()
        y_rdma.wait()
        summ[...] = raw[...] + yrecv[...]
        out_ref[pl.ds(b_mine * mb, mb), :] = summ[...].astype(jnp.float32)

        x_rdma = exchange(summ, xrecv, 1, x_peer)
        z_rdma = exchange(summ, zrecv, 2, z_peer)
        d_rdma = exchange(summ, drecv, 3, d_peer)
        x_rdma.start()
        z_rdma.start()
        d_rdma.start()
        x_rdma.wait()
        out_ref[pl.ds(b_x * mb, mb), :] = xrecv[...].astype(jnp.float32)
        z_rdma.wait()
        out_ref[pl.ds(b_z * mb, mb), :] = zrecv[...].astype(jnp.float32)
        d_rdma.wait()
        out_ref[pl.ds(b_d * mb, mb), :] = drecv[...].astype(jnp.float32)

    return pl.pallas_call(
        body,
        out_shape=jax.ShapeDtypeStruct((m, n), x.dtype),
        in_specs=[pl.BlockSpec(memory_space=pltpu.VMEM)],
        out_specs=pl.BlockSpec(memory_space=pltpu.VMEM),
        scratch_shapes=[
            pltpu.VMEM((mb, n), jnp.bfloat16),
            pltpu.VMEM((mb, n), jnp.bfloat16),
            pltpu.VMEM((mb, n), jnp.bfloat16),
            pltpu.VMEM((mb, n), jnp.bfloat16),
            pltpu.VMEM((mb, n), jnp.bfloat16),
            pltpu.VMEM((mb, n), jnp.bfloat16),
            pltpu.SemaphoreType.DMA((4,)),
            pltpu.SemaphoreType.DMA((4,)),
        ],
        compiler_params=pltpu.CompilerParams(collective_id=0),
    )(x)


# device time: 24503 ns/iter; 1.2669x vs baseline; 1.1732x over previous
import jax
import jax.numpy as jnp
from jax import lax
from jax.experimental import pallas as pl
from jax.experimental.pallas import tpu as pltpu

K = 4


def kernel(x):
    m, n = x.shape
    mb = m // 4
    cr = mb // K

    def body(x_ref, out_ref, raw, yrecv, summ, xrecv, zrecv, drecv,
             ssems, rsems):
        mx = lax.axis_index("x")
        my = lax.axis_index("y")
        mz = lax.axis_index("z")
        y_peer = (mx, 1 - my, mz)
        x_peer = (1 - mx, my, mz)
        z_peer = (mx, my, 1 - mz)
        d_peer = (1 - mx, my, 1 - mz)
        b_mine = 2 * mx + mz
        b_x = 2 * (1 - mx) + mz
        b_z = 2 * mx + (1 - mz)
        b_d = 2 * (1 - mx) + (1 - mz)

        barrier_sem = pltpu.get_barrier_semaphore()
        for p in (y_peer, x_peer, z_peer, d_peer):
            pl.semaphore_signal(
                barrier_sem, inc=1, device_id=p,
                device_id_type=pl.DeviceIdType.MESH,
            )
        pl.semaphore_wait(barrier_sem, 4)

        def exchange(src, dst, i, c, peer):
            return pltpu.make_async_remote_copy(
                src_ref=src.at[pl.ds(c * cr, cr), :],
                dst_ref=dst.at[pl.ds(c * cr, cr), :],
                send_sem=ssems.at[i, c], recv_sem=rsems.at[i, c],
                device_id=peer, device_id_type=pl.DeviceIdType.MESH,
            )

        raw[...] = x_ref[pl.ds(b_mine * mb, mb), :].astype(jnp.bfloat16)
        y_rdmas = []
        for c in range(K):
            r = exchange(raw, yrecv, 0, c, y_peer)
            r.start()
            y_rdmas.append(r)

        x_rdmas, z_rdmas, d_rdmas = [], [], []
        for c in range(K):
            cs = pl.ds(c * cr, cr)
            y_rdmas[c].wait_recv()
            summ[cs, :] = raw[cs, :] + yrecv[cs, :]
            for lst, dst, i, peer in (
                (x_rdmas, xrecv, 1, x_peer),
                (z_rdmas, zrecv, 2, z_peer),
                (d_rdmas, drecv, 3, d_peer),
            ):
                r = exchange(summ, dst, i, c, peer)
                r.start()
                lst.append(r)
            out_ref[pl.ds(b_mine * mb + c * cr, cr), :] = (
                summ[cs, :].astype(jnp.float32)
            )

        for c in range(K):
            cs = pl.ds(c * cr, cr)
            x_rdmas[c].wait_recv()
            out_ref[pl.ds(b_x * mb + c * cr, cr), :] = (
                xrecv[cs, :].astype(jnp.float32)
            )
            z_rdmas[c].wait_recv()
            out_ref[pl.ds(b_z * mb + c * cr, cr), :] = (
                zrecv[cs, :].astype(jnp.float32)
            )
            d_rdmas[c].wait_recv()
            out_ref[pl.ds(b_d * mb + c * cr, cr), :] = (
                drecv[cs, :].astype(jnp.float32)
            )

        for c in range(K):
            y_rdmas[c].wait_send()
            x_rdmas[c].wait_send()
            z_rdmas[c].wait_send()
            d_rdmas[c].wait_send()

    return pl.pallas_call(
        body,
        out_shape=jax.ShapeDtypeStruct((m, n), x.dtype),
        in_specs=[pl.BlockSpec(memory_space=pltpu.VMEM)],
        out_specs=pl.BlockSpec(memory_space=pltpu.VMEM),
        scratch_shapes=[
            pltpu.VMEM((mb, n), jnp.bfloat16),
            pltpu.VMEM((mb, n), jnp.bfloat16),
            pltpu.VMEM((mb, n), jnp.bfloat16),
            pltpu.VMEM((mb, n), jnp.bfloat16),
            pltpu.VMEM((mb, n), jnp.bfloat16),
            pltpu.VMEM((mb, n), jnp.bfloat16),
            pltpu.SemaphoreType.DMA((4, K)),
            pltpu.SemaphoreType.DMA((4, K)),
        ],
        compiler_params=pltpu.CompilerParams(collective_id=0),
    )(x)
